# baseline (speedup 1.0000x reference)
import functools

import jax
import jax.numpy as jnp
from jax import lax
from jax.experimental import pallas as pl
from jax.experimental.pallas import tpu as pltpu
from jax.experimental.pallas import tpu_sc as plsc

BATCH = 16384
EMBED_DIM = 64
HIDDEN_DIM = 256
VOCAB = 1000000
HALF_V = VOCAB // 2

NUM_CORES = 2
NUM_SUBCORES = 16
NUM_WORKERS = NUM_CORES * NUM_SUBCORES
B_PER_W = BATCH // NUM_WORKERS
LANES = 16
CHUNK = 128
N_CHUNKS = B_PER_W // CHUNK

MLP_TILE = 1024


def _gather_body(xu, xv, uid, iid, u_out, i_out,
                 uids_ref, iids_ref, uq_ref, iq_ref, rows_ref, sem):
    wid = lax.axis_index("s") * NUM_CORES + lax.axis_index("c")
    base = pl.multiple_of(wid * B_PER_W, B_PER_W)
    pltpu.sync_copy(uid.at[pl.ds(base, B_PER_W)], uids_ref)
    pltpu.sync_copy(iid.at[pl.ds(base, B_PER_W)], iids_ref)
    for g in range(B_PER_W // LANES):
        vu = uids_ref[pl.ds(g * LANES, LANES)]
        vi = iids_ref[pl.ds(g * LANES, LANES)]
        uq_ref[pl.ds(g * LANES, LANES)] = vu - jnp.where(
            vu >= HALF_V, HALF_V, 0).astype(jnp.int32)
        iq_ref[pl.ds(g * LANES, LANES)] = vi - jnp.where(
            vi >= HALF_V, HALF_V, 0).astype(jnp.int32)
    for table, q_ref, out in ((xu, uq_ref, u_out), (xv, iq_ref, i_out)):
        copies = []
        for k in range(N_CHUNKS):
            copies.append(pltpu.async_copy(
                table.at[q_ref.at[pl.ds(k * CHUNK, CHUNK)]],
                rows_ref.at[pl.ds(k * CHUNK, CHUNK)], sem))
        for c in copies:
            c.wait()
        pltpu.sync_copy(rows_ref, out.at[pl.ds(base, B_PER_W)])


def _sc_gather(xu, xv, user_id, item_id):
    emb = jax.ShapeDtypeStruct((BATCH, 2 * EMBED_DIM), jnp.float32)
    fn = functools.partial(
        pl.kernel,
        mesh=plsc.VectorSubcoreMesh(core_axis_name="c", subcore_axis_name="s"),
        out_type=(emb, emb),
        scratch_types=[
            pltpu.VMEM((B_PER_W,), jnp.int32),
            pltpu.VMEM((B_PER_W,), jnp.int32),
            pltpu.VMEM((B_PER_W,), jnp.int32),
            pltpu.VMEM((B_PER_W,), jnp.int32),
            pltpu.VMEM((B_PER_W, 2 * EMBED_DIM), jnp.float32),
            pltpu.SemaphoreType.DMA,
        ],
    )(_gather_body)
    return fn(xu, xv, user_id, item_id)


def _mlp_body(xu_ref, xi_ref, uid_ref, iid_ref, xf_ref, w1u_ref, w1i_ref,
              w1f_ref, b1_ref, w2_ref, b2_ref, o_ref):
    lane_half = lax.broadcasted_iota(jnp.int32, (MLP_TILE, 2 * EMBED_DIM),
                                     1) >= EMBED_DIM
    usel = (uid_ref[...] >= HALF_V)
    isel = (iid_ref[...] >= HALF_V)
    xu = jnp.where(lane_half == usel, xu_ref[...], 0.0)
    xi = jnp.where(lane_half == isel, xi_ref[...], 0.0)
    h = jnp.dot(xu, w1u_ref[...], preferred_element_type=jnp.float32)
    h = h + jnp.dot(xi, w1i_ref[...], preferred_element_type=jnp.float32)
    h = h + jnp.dot(xf_ref[...], w1f_ref[...],
                    preferred_element_type=jnp.float32)
    h = jnp.maximum(h + b1_ref[...], 0.0)
    y = jnp.dot(h, w2_ref[...], preferred_element_type=jnp.float32) + b2_ref[...]
    o_ref[...] = jax.nn.sigmoid(y)


def _tc_mlp(xu, xi, uid2, iid2, xf, W1, b1, W2, b2):
    w1u = jnp.concatenate([W1[:EMBED_DIM]] * 2, axis=0)
    w1i = jnp.concatenate([W1[EMBED_DIM:2 * EMBED_DIM]] * 2, axis=0)
    w1f = W1[2 * EMBED_DIM:]
    b1_2d = b1.reshape(1, HIDDEN_DIM)
    b2_2d = b2.reshape(1, 1)
    grid = BATCH // MLP_TILE
    out = pl.pallas_call(
        _mlp_body,
        grid=(grid,),
        in_specs=[
            pl.BlockSpec((MLP_TILE, 2 * EMBED_DIM), lambda t: (t, 0)),
            pl.BlockSpec((MLP_TILE, 2 * EMBED_DIM), lambda t: (t, 0)),
            pl.BlockSpec((MLP_TILE, 1), lambda t: (t, 0)),
            pl.BlockSpec((MLP_TILE, 1), lambda t: (t, 0)),
            pl.BlockSpec((MLP_TILE, 2), lambda t: (t, 0)),
            pl.BlockSpec((2 * EMBED_DIM, HIDDEN_DIM), lambda t: (0, 0)),
            pl.BlockSpec((2 * EMBED_DIM, HIDDEN_DIM), lambda t: (0, 0)),
            pl.BlockSpec((2, HIDDEN_DIM), lambda t: (0, 0)),
            pl.BlockSpec((1, HIDDEN_DIM), lambda t: (0, 0)),
            pl.BlockSpec((HIDDEN_DIM, 1), lambda t: (0, 0)),
            pl.BlockSpec((1, 1), lambda t: (0, 0)),
        ],
        out_specs=pl.BlockSpec((MLP_TILE, 1), lambda t: (t, 0)),
        out_shape=jax.ShapeDtypeStruct((BATCH, 1), jnp.float32),
    )(xu, xi, uid2, iid2, xf, w1u, w1i, w1f, b1_2d, W2, b2_2d)
    return out[:, 0]


def _repack(table):
    return jnp.concatenate([table[:HALF_V], table[HALF_V:]], axis=1)


def kernel(user_id, item_id, user_feature, item_feature, user_table,
           item_table, W1, b1, W2, b2):
    xu_t = _repack(user_table)
    xv_t = _repack(item_table)
    xu, xi = _sc_gather(xu_t, xv_t, user_id, item_id)
    xf = jnp.stack([user_feature, item_feature], axis=1)
    return _tc_mlp(xu, xi, user_id.reshape(BATCH, 1), item_id.reshape(BATCH, 1),
                   xf, W1, b1, W2, b2)


# speedup vs baseline: 1.0319x; 1.0319x over previous
import functools

import jax
import jax.numpy as jnp
from jax import lax
from jax.experimental import pallas as pl
from jax.experimental.pallas import tpu as pltpu
from jax.experimental.pallas import tpu_sc as plsc

BATCH = 16384
EMBED_DIM = 64
HIDDEN_DIM = 256
VOCAB = 1000000
XROWS = 1000448

NUM_CORES = 2
NUM_SUBCORES = 16
NUM_WORKERS = NUM_CORES * NUM_SUBCORES
B_PER_W = BATCH // NUM_WORKERS
LANES = 16
CHUNK = 128
N_CHUNKS = B_PER_W // CHUNK

MLP_TILE = 1024


def _gather_body(xu, xv, uid, iid, u_out, i_out,
                 uids_ref, iids_ref, rows_ref, sem):
    wid = lax.axis_index("s") * NUM_CORES + lax.axis_index("c")
    base = pl.multiple_of(wid * B_PER_W, B_PER_W)
    pltpu.sync_copy(uid.at[pl.ds(base, B_PER_W)], uids_ref)
    pltpu.sync_copy(iid.at[pl.ds(base, B_PER_W)], iids_ref)
    for table, q_ref, out in ((xu, uids_ref, u_out), (xv, iids_ref, i_out)):
        copies = []
        for k in range(N_CHUNKS):
            copies.append(pltpu.async_copy(
                table.at[q_ref.at[pl.ds(k * CHUNK, CHUNK)]],
                rows_ref.at[pl.ds(k * CHUNK, CHUNK)], sem))
        for c in copies:
            c.wait()
        pltpu.sync_copy(rows_ref, out.at[pl.ds(base, B_PER_W)])


def _sc_gather(xu, xv, user_id, item_id):
    emb = jax.ShapeDtypeStruct((BATCH, 2 * EMBED_DIM), jnp.float32)
    fn = functools.partial(
        pl.kernel,
        mesh=plsc.VectorSubcoreMesh(core_axis_name="c", subcore_axis_name="s"),
        out_type=(emb, emb),
        scratch_types=[
            pltpu.VMEM((B_PER_W,), jnp.int32),
            pltpu.VMEM((B_PER_W,), jnp.int32),
            pltpu.VMEM((B_PER_W, 2 * EMBED_DIM), jnp.float32),
            pltpu.SemaphoreType.DMA,
        ],
    )(_gather_body)
    return fn(xu, xv, user_id, item_id)


def _mlp_body(xu_ref, xi_ref, xf_ref, w1u_ref, w1i_ref,
              w1f_ref, b1_ref, w2_ref, b2_ref, o_ref):
    h = jnp.dot(xu_ref[...], w1u_ref[...], preferred_element_type=jnp.float32)
    h = h + jnp.dot(xi_ref[...], w1i_ref[...], preferred_element_type=jnp.float32)
    h = h + jnp.dot(xf_ref[...], w1f_ref[...],
                    preferred_element_type=jnp.float32)
    h = jnp.maximum(h + b1_ref[...], 0.0)
    y = jnp.dot(h, w2_ref[...], preferred_element_type=jnp.float32) + b2_ref[...]
    o_ref[...] = jax.nn.sigmoid(y)


def _tc_mlp(xu, xi, xf, W1, b1, W2, b2):
    zpad = jnp.zeros((EMBED_DIM, HIDDEN_DIM), jnp.float32)
    w1u = jnp.concatenate([W1[:EMBED_DIM], zpad], axis=0)
    w1i = jnp.concatenate([W1[EMBED_DIM:2 * EMBED_DIM], zpad], axis=0)
    w1f = W1[2 * EMBED_DIM:]
    b1_2d = b1.reshape(1, HIDDEN_DIM)
    b2_2d = b2.reshape(1, 1)
    grid = BATCH // MLP_TILE
    out = pl.pallas_call(
        _mlp_body,
        grid=(grid,),
        in_specs=[
            pl.BlockSpec((MLP_TILE, 2 * EMBED_DIM), lambda t: (t, 0)),
            pl.BlockSpec((MLP_TILE, 2 * EMBED_DIM), lambda t: (t, 0)),
            pl.BlockSpec((MLP_TILE, 2), lambda t: (t, 0)),
            pl.BlockSpec((2 * EMBED_DIM, HIDDEN_DIM), lambda t: (0, 0)),
            pl.BlockSpec((2 * EMBED_DIM, HIDDEN_DIM), lambda t: (0, 0)),
            pl.BlockSpec((2, HIDDEN_DIM), lambda t: (0, 0)),
            pl.BlockSpec((1, HIDDEN_DIM), lambda t: (0, 0)),
            pl.BlockSpec((HIDDEN_DIM, 1), lambda t: (0, 0)),
            pl.BlockSpec((1, 1), lambda t: (0, 0)),
        ],
        out_specs=pl.BlockSpec((MLP_TILE, 1), lambda t: (t, 0)),
        out_shape=jax.ShapeDtypeStruct((BATCH, 1), jnp.float32),
    )(xu, xi, xf, w1u, w1i, w1f, b1_2d, W2, b2_2d)
    return out[:, 0]


REPACK_BLK = 1024


def _repack_body(in1_ref, o_ref):
    left = jnp.transpose(in1_ref[...], (1, 0))
    o_ref[...] = jnp.concatenate(
        [left, jnp.zeros((REPACK_BLK, EMBED_DIM), jnp.float32)], axis=1)


def _repack(table_t):
    grid = XROWS // REPACK_BLK
    return pl.pallas_call(
        _repack_body,
        grid=(grid,),
        in_specs=[pl.BlockSpec((EMBED_DIM, REPACK_BLK), lambda g: (0, g))],
        out_specs=pl.BlockSpec((REPACK_BLK, 2 * EMBED_DIM), lambda g: (g, 0)),
        out_shape=jax.ShapeDtypeStruct((XROWS, 2 * EMBED_DIM), jnp.float32),
    )(table_t)


def kernel(user_id, item_id, user_feature, item_feature, user_table,
           item_table, W1, b1, W2, b2):
    xu_t = _repack(user_table.T)
    xv_t = _repack(item_table.T)
    xu, xi = _sc_gather(xu_t, xv_t, user_id, item_id)
    xf = jnp.stack([user_feature, item_feature], axis=1)
    return _tc_mlp(xu, xi, xf, W1, b1, W2, b2)
